# baseline (speedup 1.0000x reference)
import functools

import jax
import jax.numpy as jnp
from jax import lax
from jax.experimental import pallas as pl
from jax.experimental.pallas import tpu as pltpu, tpu_sc as plsc

N = 16777216
_INFO = plsc.get_sparse_core_info()
NC = _INFO.num_cores
NS = _INFO.num_subcores
L = _INFO.num_lanes
NW = NC * NS
PER_W = N // NW
CHUNK = 8192
NCHUNK = PER_W // CHUNK
DEPTH = 4

_mesh = plsc.VectorSubcoreMesh(core_axis_name="c", subcore_axis_name="s")


@functools.partial(
    pl.kernel,
    out_type=jax.ShapeDtypeStruct((N,), jnp.float32),
    mesh=_mesh,
    scratch_types=[
        [pltpu.VMEM((CHUNK,), jnp.float32) for _ in range(DEPTH)],
        [pltpu.VMEM((CHUNK,), jnp.float32) for _ in range(DEPTH)],
        [pltpu.VMEM((CHUNK,), jnp.float32) for _ in range(DEPTH)],
        [pltpu.SemaphoreType.DMA for _ in range(DEPTH)],
        [pltpu.SemaphoreType.DMA for _ in range(DEPTH)],
        [pltpu.SemaphoreType.DMA for _ in range(DEPTH)],
    ],
)
def _mod_add_sc(a_hbm, b_hbm, out_hbm, a_bufs, b_bufs, o_bufs,
                ina_sems, inb_sems, out_sems):
    wid = lax.axis_index("s") * NC + lax.axis_index("c")
    base = wid * PER_W

    def start_in(c, s):
        off = base + c * CHUNK
        pltpu.make_async_copy(
            a_hbm.at[pl.ds(off, CHUNK)], a_bufs[s], ina_sems[s]).start()
        pltpu.make_async_copy(
            b_hbm.at[pl.ds(off, CHUNK)], b_bufs[s], inb_sems[s]).start()

    def wait_in(s):
        pltpu.make_async_copy(
            a_hbm.at[pl.ds(base, CHUNK)], a_bufs[s], ina_sems[s]).wait()
        pltpu.make_async_copy(
            b_hbm.at[pl.ds(base, CHUNK)], b_bufs[s], inb_sems[s]).wait()

    def start_out(c, s):
        off = base + c * CHUNK
        pltpu.make_async_copy(
            o_bufs[s], out_hbm.at[pl.ds(off, CHUNK)], out_sems[s]).start()

    def wait_out(s):
        pltpu.make_async_copy(
            o_bufs[s], out_hbm.at[pl.ds(base, CHUNK)], out_sems[s]).wait()

    for s in range(DEPTH - 1):
        start_in(s, s)

    def ring_body(p, carry):
        for s in range(DEPTH):
            c = p * DEPTH + s

            @pl.when(c + DEPTH - 1 < NCHUNK)
            def _():
                start_in(c + DEPTH - 1, (s + DEPTH - 1) % DEPTH)

            wait_in(s)

            @pl.when(c >= DEPTH)
            def _():
                wait_out(s)

            a_buf, b_buf, o_buf = a_bufs[s], b_bufs[s], o_bufs[s]

            @plsc.parallel_loop(0, CHUNK, step=L, unroll=8)
            def _(j):
                av = a_buf[pl.ds(j, L)]
                bv = b_buf[pl.ds(j, L)]
                v = av + bv
                v = jnp.where(v >= 256.0, v - 256.0, v)
                v = jnp.where(v >= 256.0, v - 256.0, v)
                o_buf[pl.ds(j, L)] = v

            start_out(c, s)
        return carry

    lax.fori_loop(0, NCHUNK // DEPTH, ring_body, 0)
    for s in range(DEPTH):
        wait_out(s)


def kernel(a, b):
    out = _mod_add_sc(a, b)
    return out[:, None]
